# Initial kernel scaffold; baseline (speedup 1.0000x reference)
import jax
import jax.numpy as jnp
from jax.experimental import pallas as pl


def kernel(inputs, embedding_weight):
    raise NotImplementedError("write your pallas kernel here")


# speedup vs baseline: 1.0554x; 1.0554x over previous
import functools

import jax
import jax.numpy as jnp
from jax import lax
from jax.experimental import pallas as pl
from jax.experimental.pallas import tpu as pltpu
from jax.experimental.pallas import tpu_sc as plsc

NUM_CODES = 8192
DIM = 256
N_TOKENS = 8192
COMMIT = 0.25


_BN = 1024
_BK = 1024


def _argmin_body(x_ref, e_ref, idx_ref, minv_ref):
    j = pl.program_id(1)
    x = x_ref[...]
    e = e_ref[...]
    mm = lax.dot_general(x, e, (((1,), (1,)), ((), ())),
                         preferred_element_type=jnp.float32)
    xn = jnp.sum(x * x, axis=1, keepdims=True)
    en = jnp.sum(e * e, axis=1)
    dist = (xn + en[None, :]) - 2.0 * mm
    m = jnp.min(dist, axis=1, keepdims=True)
    col = lax.broadcasted_iota(jnp.int32, dist.shape, 1) + j * _BK
    lidx = jnp.min(jnp.where(dist == m, col, jnp.int32(NUM_CODES)),
                   axis=1, keepdims=True)

    @pl.when(j == 0)
    def _():
        minv_ref[...] = m
        idx_ref[...] = lidx

    @pl.when(j != 0)
    def _():
        better = m < minv_ref[...]
        idx_ref[...] = jnp.where(better, lidx, idx_ref[...])
        minv_ref[...] = jnp.where(better, m, minv_ref[...])


def _run_argmin(flat_x, emb):
    return pl.pallas_call(
        _argmin_body,
        grid=(N_TOKENS // _BN, NUM_CODES // _BK),
        in_specs=[
            pl.BlockSpec((_BN, DIM), lambda i, j: (i, 0)),
            pl.BlockSpec((_BK, DIM), lambda i, j: (j, 0)),
        ],
        out_specs=pl.BlockSpec((_BN, 1), lambda i, j: (i, 0)),
        out_shape=jax.ShapeDtypeStruct((N_TOKENS, 1), jnp.int32),
        scratch_shapes=[pltpu.VMEM((_BN, 1), jnp.float32)],
    )(flat_x, emb)


_BN2 = 512
_BK2 = 2048


def _onehot_body(idx_ref, out_ref):
    j = pl.program_id(1)
    col = lax.broadcasted_iota(jnp.int32, (_BN2, _BK2), 1) + j * _BK2
    out_ref[...] = (col == idx_ref[...]).astype(jnp.float32)


def _run_onehot(idx2d):
    return pl.pallas_call(
        _onehot_body,
        grid=(N_TOKENS // _BN2, NUM_CODES // _BK2),
        in_specs=[pl.BlockSpec((_BN2, 1), lambda i, j: (i, 0))],
        out_specs=pl.BlockSpec((_BN2, _BK2), lambda i, j: (i, j)),
        out_shape=jax.ShapeDtypeStruct((N_TOKENS, NUM_CODES), jnp.float32),
    )(idx2d)


def _make_sc_gather():
    info = plsc.get_sparse_core_info()
    nc, ns, nl = info.num_cores, info.num_subcores, info.num_lanes
    nw = nc * ns
    b_per_w = N_TOKENS // nw
    ch = 128
    nchunks = b_per_w // ch
    mesh = plsc.VectorSubcoreMesh(core_axis_name="c", subcore_axis_name="s")

    @functools.partial(
        pl.kernel,
        mesh=mesh,
        out_type=[
            jax.ShapeDtypeStruct((N_TOKENS, DIM), jnp.float32),
            jax.ShapeDtypeStruct((nw, nl), jnp.float32),
        ],
        scratch_types=[
            pltpu.VMEM((ch,), jnp.int32),
            pltpu.VMEM((ch, DIM), jnp.float32),
            pltpu.VMEM((ch, DIM), jnp.float32),
            pltpu.VMEM((nl,), jnp.float32),
            pltpu.SemaphoreType.DMA,
        ],
    )
    def sc_gather(table_hbm, idx_hbm, x_hbm, out_hbm, loss_hbm,
                  idx_v, rows_v, x_v, acc_v, sem):
        wid = lax.axis_index("s") * nc + lax.axis_index("c")
        base = wid * b_per_w
        acc = jnp.zeros((nl,), jnp.float32)
        for cidx in range(nchunks):
            cb = base + cidx * ch
            pltpu.sync_copy(idx_hbm.at[pl.ds(cb, ch)], idx_v)
            pltpu.async_copy(table_hbm.at[idx_v], rows_v, sem).wait()
            pltpu.sync_copy(x_hbm.at[pl.ds(cb, ch)], x_v)

            def row_body(r, acc):
                for c in range(DIM // nl):
                    sl = pl.ds(c * nl, nl)
                    xv = x_v[r, sl]
                    qv = rows_v[r, sl]
                    dv = qv - xv
                    rows_v[r, sl] = xv + dv
                    acc = acc + dv * dv
                return acc

            acc = lax.fori_loop(0, ch, row_body, acc)
            pltpu.sync_copy(rows_v, out_hbm.at[pl.ds(cb, ch)])
        acc_v[...] = acc
        pltpu.sync_copy(acc_v, loss_hbm.at[wid])

    return sc_gather


_sc_gather = None


def kernel(inputs, embedding_weight):
    global _sc_gather
    if _sc_gather is None:
        _sc_gather = _make_sc_gather()
    seqlen, bs, d = inputs.shape
    flat = inputs.reshape(-1, d)
    idx2d = _run_argmin(flat, embedding_weight)
    encodings = _run_onehot(idx2d)
    st_flat, partials = _sc_gather(embedding_weight, idx2d.reshape(-1), flat)
    loss = COMMIT * (jnp.sum(partials) / jnp.float32(N_TOKENS * DIM))
    return (
        loss,
        st_flat.reshape(seqlen, bs, d),
        encodings.reshape(seqlen, bs, NUM_CODES),
        idx2d,
    )
